# baseline (speedup 1.0000x reference)
import functools

import jax
import jax.numpy as jnp
from jax import lax
from jax.experimental import pallas as pl
from jax.experimental.pallas import tpu as pltpu
from jax.experimental.pallas import tpu_sc as plsc

BATCH = 16384
D = 64
H1 = 128
H2 = 64
NC = 2
NS = 16
NW = NC * NS
BPW = BATCH // NW
CHUNK = 128
K = BPW // CHUNK

BT = 2048


def _sc_gather(user_table, item_table, uid3, iid3):
    mesh = plsc.VectorSubcoreMesh(core_axis_name="c", subcore_axis_name="s")

    @functools.partial(
        pl.kernel,
        mesh=mesh,
        compiler_params=pltpu.CompilerParams(use_tc_tiling_on_sc=False),
        out_type=[
            jax.ShapeDtypeStruct((BATCH, D), jnp.float32),
            jax.ShapeDtypeStruct((BATCH, D), jnp.float32),
        ],
        scratch_types=[
            pltpu.VMEM((K, CHUNK), jnp.int32),
            pltpu.VMEM((K, CHUNK), jnp.int32),
            pltpu.VMEM((BPW, D), jnp.float32),
            pltpu.VMEM((BPW, D), jnp.float32),
            pltpu.SemaphoreType.DMA,
        ],
    )
    def k(ut_hbm, it_hbm, uid_hbm, iid_hbm, uo_hbm, io_hbm,
          uidx_v, iidx_v, urows_v, irows_v, sem):
        wid = lax.axis_index("s") * NC + lax.axis_index("c")
        pltpu.sync_copy(uid_hbm.at[wid], uidx_v)
        pltpu.sync_copy(iid_hbm.at[wid], iidx_v)
        copies = []
        for j in range(K):
            dst = pl.ds(j * CHUNK, CHUNK)
            copies.append(
                pltpu.async_copy(ut_hbm.at[uidx_v.at[j]], urows_v.at[dst], sem))
            copies.append(
                pltpu.async_copy(it_hbm.at[iidx_v.at[j]], irows_v.at[dst], sem))
        for c in copies:
            c.wait()
        base = pl.ds(wid * BPW, BPW)
        pltpu.sync_copy(urows_v, uo_hbm.at[base])
        pltpu.sync_copy(irows_v, io_hbm.at[base])

    return k(user_table, item_table, uid3, iid3)


def _mlp_body(ue_ref, ie_ref, w1u_ref, w1i_ref, b1_ref, w2_ref, b2_ref,
              w3t_ref, b3_ref, out_ref):
    h = jnp.dot(ue_ref[...], w1u_ref[...], preferred_element_type=jnp.float32)
    h = h + jnp.dot(ie_ref[...], w1i_ref[...],
                    preferred_element_type=jnp.float32)
    h = jax.nn.relu(h + b1_ref[...])
    h = jax.nn.relu(jnp.dot(h, w2_ref[...],
                            preferred_element_type=jnp.float32) + b2_ref[...])
    p = jnp.sum(h * w3t_ref[...], axis=1, keepdims=True) + b3_ref[...]
    out_ref[...] = jax.nn.sigmoid(p)


def _tc_mlp(ue, ie, W1, b1, W2, b2, W3, b3):
    w1u = W1[:D]
    w1i = W1[D:]
    b1r = b1.reshape(1, H1)
    b2r = b2.reshape(1, H2)
    w3t = W3.reshape(1, H2)
    b3r = b3.reshape(1, 1)
    rep = lambda i: (0, 0)
    out = pl.pallas_call(
        _mlp_body,
        grid=(BATCH // BT,),
        in_specs=[
            pl.BlockSpec((BT, D), lambda i: (i, 0)),
            pl.BlockSpec((BT, D), lambda i: (i, 0)),
            pl.BlockSpec((D, H1), rep),
            pl.BlockSpec((D, H1), rep),
            pl.BlockSpec((1, H1), rep),
            pl.BlockSpec((H1, H2), rep),
            pl.BlockSpec((1, H2), rep),
            pl.BlockSpec((1, H2), rep),
            pl.BlockSpec((1, 1), rep),
        ],
        out_specs=pl.BlockSpec((BT, 1), lambda i: (i, 0)),
        out_shape=jax.ShapeDtypeStruct((BATCH, 1), jnp.float32),
    )(ue, ie, w1u, w1i, b1r, W2, b2r, w3t, b3r)
    return out.reshape(BATCH)


def kernel(user_ids, item_ids, user_table, item_table, W1, b1, W2, b2, W3, b3):
    uid3 = user_ids.astype(jnp.int32).reshape(NW, K, CHUNK)
    iid3 = item_ids.astype(jnp.int32).reshape(NW, K, CHUNK)
    ue, ie = _sc_gather(user_table, item_table, uid3, iid3)
    return _tc_mlp(ue, ie, W1, b1, W2, b2, W3, b3)
